# baseline (speedup 1.0000x reference)
import jax
import jax.numpy as jnp
from jax.experimental import pallas as pl
from jax.experimental.pallas import tpu as pltpu

_B, _S, _H = 4, 4096, 1024
_SLOTS = 1024
_TOP_K = 16
_TBLK = 1024
_HBLK = 256
_RSTRIP = 32
_NCHUNK = 8
_LANES = _SLOTS // _NCHUNK
_NEG = -1e30

_SORT8 = [(0, 1), (2, 3), (4, 5), (6, 7),
          (0, 2), (1, 3), (4, 6), (5, 7),
          (1, 2), (5, 6), (0, 4), (3, 7),
          (1, 5), (2, 6),
          (1, 4), (3, 6),
          (2, 4), (3, 5),
          (3, 4)]


def _topk_threshold(s):
    t = [s[:, c * _LANES:(c + 1) * _LANES] for c in range(_NCHUNK)]
    for a, b in _SORT8:
        hi = jnp.maximum(t[a], t[b])
        lo = jnp.minimum(t[a], t[b])
        t[a], t[b] = hi, lo
    rowmax = None
    m = None
    for k in range(_TOP_K):
        m = jnp.max(t[0], axis=-1, keepdims=True)
        if k == 0:
            rowmax = m
        if k == _TOP_K - 1:
            break
        mask = t[0] >= m
        for j in range(_NCHUNK - 1):
            t[j] = jnp.where(mask, t[j + 1], t[j])
        t[_NCHUNK - 1] = jnp.where(mask, _NEG, t[_NCHUNK - 1])
    return rowmax, m


def _lava_body(x_ref, waddr_ref, addrt_ref, contents_ref, wread_ref,
               out_ref, anorm_ref):
    i = pl.program_id(0)

    @pl.when(i == 0)
    def _():
        a_t = addrt_ref[...]
        norm = jnp.sqrt(jnp.sum(a_t * a_t, axis=0, keepdims=True))
        anorm_ref[...] = a_t / jnp.maximum(norm, 1e-8)

    def _scores(h0):
        xb = x_ref[h0:h0 + _HBLK, :]
        q = jnp.dot(xb, waddr_ref[...], preferred_element_type=jnp.float32)
        qn = q / jnp.maximum(
            jnp.sqrt(jnp.sum(q * q, axis=-1, keepdims=True)), 1e-6)
        return jnp.dot(qn, anorm_ref[...], preferred_element_type=jnp.float32)

    def _attn(scores):
        attn_parts = []
        for r0 in range(0, _HBLK, _RSTRIP):
            s = scores[r0:r0 + _RSTRIP, :]
            rowmax, thr = _topk_threshold(s)
            e = jnp.where(s >= thr, jnp.exp(s - rowmax), 0.0)
            attn_parts.append(
                (e / jnp.sum(e, axis=-1, keepdims=True)).astype(jnp.bfloat16))
        return jnp.concatenate(attn_parts, axis=0)

    def _write_out(h0, attn):
        mem = jnp.dot(attn, contents_ref[...],
                      preferred_element_type=jnp.float32)
        out_ref[h0:h0 + _HBLK, :] = jnp.dot(
            mem.astype(jnp.bfloat16), wread_ref[...],
            preferred_element_type=jnp.float32)

    n_chains = _TBLK // _HBLK
    s = [None] * n_chains
    s[0] = _scores(0)
    for c in range(n_chains):
        if c + 1 < n_chains:
            s[c + 1] = _scores((c + 1) * _HBLK)
        a = _attn(s[c])
        _write_out(c * _HBLK, a)


def kernel(x, W_addr, W_read, addresses, contents):
    n = _B * _S
    x_flat = x.reshape(n, _H)
    grid = (n // _TBLK,)
    out = pl.pallas_call(
        _lava_body,
        grid=grid,
        in_specs=[
            pl.BlockSpec((_TBLK, _H), lambda i: (i, 0)),
            pl.BlockSpec((_H, _H), lambda i: (0, 0)),
            pl.BlockSpec((_H, _SLOTS), lambda i: (0, 0)),
            pl.BlockSpec((_SLOTS, _H), lambda i: (0, 0)),
            pl.BlockSpec((_H, _H), lambda i: (0, 0)),
        ],
        out_specs=pl.BlockSpec((_TBLK, _H), lambda i: (i, 0)),
        out_shape=jax.ShapeDtypeStruct((n, _H), jnp.float32),
        scratch_shapes=[pltpu.VMEM((_H, _SLOTS), jnp.float32)],
    )(x_flat, W_addr.T, addresses.T, contents.astype(jnp.bfloat16),
      W_read.T.astype(jnp.bfloat16))
    return out.reshape(_B, _S, _H)


# speedup vs baseline: 1.2166x; 1.2166x over previous
import jax
import jax.numpy as jnp
from jax.experimental import pallas as pl
from jax.experimental.pallas import tpu as pltpu

_B, _S, _H = 4, 4096, 1024
_SLOTS = 1024
_TOP_K = 16
_TBLK = 1024
_HBLK = 512
_RSTRIP = 32
_NCHUNK = 8
_LANES = _SLOTS // _NCHUNK
_NEG = -1e30

_SORT8 = [(0, 1), (2, 3), (4, 5), (6, 7),
          (0, 2), (1, 3), (4, 6), (5, 7),
          (1, 2), (5, 6), (0, 4), (3, 7),
          (1, 5), (2, 6),
          (1, 4), (3, 6),
          (2, 4), (3, 5),
          (3, 4)]


def _topk_threshold(s):
    t = [s[:, c * _LANES:(c + 1) * _LANES] for c in range(_NCHUNK)]
    for a, b in _SORT8:
        hi = jnp.maximum(t[a], t[b])
        lo = jnp.minimum(t[a], t[b])
        t[a], t[b] = hi, lo
    rowmax = None
    m = None
    for k in range(_TOP_K):
        m = jnp.max(t[0], axis=-1, keepdims=True)
        if k == 0:
            rowmax = m
        if k == _TOP_K - 1:
            break
        mask = t[0] >= m
        for j in range(_NCHUNK - 1):
            t[j] = jnp.where(mask, t[j + 1], t[j])
        t[_NCHUNK - 1] = jnp.where(mask, _NEG, t[_NCHUNK - 1])
    return rowmax, m


def _lava_body(x_ref, waddr_ref, waddrbf_ref, addrt_ref, contents_ref,
               wread_ref, out_ref, m_ref, c2_ref):
    i = pl.program_id(0)

    @pl.when(i == 0)
    def _():
        a_t = addrt_ref[...]
        norm = jnp.sqrt(jnp.sum(a_t * a_t, axis=0, keepdims=True))
        an = a_t / jnp.maximum(norm, 1e-8)
        m_ref[...] = jnp.dot(waddr_ref[...], an,
                             preferred_element_type=jnp.float32)
        c2_ref[...] = jnp.dot(contents_ref[...], wread_ref[...],
                              preferred_element_type=jnp.float32
                              ).astype(jnp.bfloat16)

    def _scores(h0):
        xb = x_ref[h0:h0 + _HBLK, :]
        qb = jnp.dot(xb.astype(jnp.bfloat16), waddrbf_ref[...],
                     preferred_element_type=jnp.float32)
        rn = 1.0 / jnp.maximum(
            jnp.sqrt(jnp.sum(qb * qb, axis=-1, keepdims=True)), 1e-6)
        su = jnp.dot(xb, m_ref[...], preferred_element_type=jnp.float32)
        return su, rn

    def _attn(scores):
        su, rn = scores
        attn_parts = []
        for r0 in range(0, _HBLK, _RSTRIP):
            s = su[r0:r0 + _RSTRIP, :]
            rowmax, thr = _topk_threshold(s)
            e = jnp.where(s >= thr,
                          jnp.exp((s - rowmax) * rn[r0:r0 + _RSTRIP, :]),
                          0.0)
            attn_parts.append(
                (e / jnp.sum(e, axis=-1, keepdims=True)).astype(jnp.bfloat16))
        return jnp.concatenate(attn_parts, axis=0)

    def _write_out(h0, attn):
        out_ref[h0:h0 + _HBLK, :] = jnp.dot(
            attn, c2_ref[...], preferred_element_type=jnp.float32)

    n_chains = _TBLK // _HBLK
    s = [None] * n_chains
    s[0] = _scores(0)
    for c in range(n_chains):
        if c + 1 < n_chains:
            s[c + 1] = _scores((c + 1) * _HBLK)
        a = _attn(s[c])
        _write_out(c * _HBLK, a)


def kernel(x, W_addr, W_read, addresses, contents):
    n = _B * _S
    x_flat = x.reshape(n, _H)
    grid = (n // _TBLK,)
    w_addr_t = W_addr.T
    out = pl.pallas_call(
        _lava_body,
        grid=grid,
        in_specs=[
            pl.BlockSpec((_TBLK, _H), lambda i: (i, 0)),
            pl.BlockSpec((_H, _H), lambda i: (0, 0)),
            pl.BlockSpec((_H, _H), lambda i: (0, 0)),
            pl.BlockSpec((_H, _SLOTS), lambda i: (0, 0)),
            pl.BlockSpec((_SLOTS, _H), lambda i: (0, 0)),
            pl.BlockSpec((_H, _H), lambda i: (0, 0)),
        ],
        out_specs=pl.BlockSpec((_TBLK, _H), lambda i: (i, 0)),
        out_shape=jax.ShapeDtypeStruct((n, _H), jnp.float32),
        scratch_shapes=[pltpu.VMEM((_H, _SLOTS), jnp.float32),
                        pltpu.VMEM((_SLOTS, _H), jnp.bfloat16)],
    )(x_flat, w_addr_t, w_addr_t.astype(jnp.bfloat16), addresses.T,
      contents.astype(jnp.bfloat16), W_read.T.astype(jnp.bfloat16))
    return out.reshape(_B, _S, _H)
